# Initial kernel scaffold; baseline (speedup 1.0000x reference)
import jax
import jax.numpy as jnp
from jax.experimental import pallas as pl


def kernel(input_ids, token_type_ids, position_ids, word_emb, pos_emb, type_emb, ln_gamma, ln_beta):
    raise NotImplementedError("write your pallas kernel here")


# speedup vs baseline: 1.3685x; 1.3685x over previous
import functools

import jax
import jax.numpy as jnp
from jax import lax
from jax.experimental import pallas as pl
from jax.experimental.pallas import tpu as pltpu
from jax.experimental.pallas import tpu_sc as plsc

HIDDEN = 768
NSL = HIDDEN // 16
EPS = 1e-12
TB = 16
T_CHUNK = 64


def _rsqrt16(x):
    i = plsc.bitcast(x, jnp.int32)
    i = jnp.int32(0x5F3759DF) - (i >> 1)
    y = plsc.bitcast(i, jnp.float32)
    for _ in range(4):
        y = y * (1.5 - 0.5 * x * y * y)
    return y


def _sc_body(ids_h, pids_h, tids_h, wemb_h, pemb_h, temb_h, gam_h, bet_h,
             out_h, idw_v, idp_v, idt_v, w_v, p_v, ty_v, g_v, b_v,
             s1_v, s2_v, sem_w, sem_p):
    info = plsc.get_sparse_core_info()
    nw = info.num_cores * info.num_subcores
    wid = lax.axis_index("s") * info.num_cores + lax.axis_index("c")
    total = ids_h.shape[0]
    per_w = total // nw
    n_chunks = per_w // T_CHUNK
    base = wid * per_w

    pltpu.sync_copy(temb_h, ty_v)
    pltpu.sync_copy(gam_h, g_v)
    pltpu.sync_copy(bet_h, b_v)

    inv_h = 1.0 / HIDDEN

    def chunk_body(ci, _):
        off = base + ci * T_CHUNK
        pltpu.sync_copy(ids_h.at[pl.ds(off, T_CHUNK)], idw_v)
        pltpu.sync_copy(pids_h.at[pl.ds(off, T_CHUNK)], idp_v)
        pltpu.sync_copy(tids_h.at[pl.ds(off, T_CHUNK)], idt_v)
        cpw = pltpu.async_copy(wemb_h.at[idw_v], w_v, sem_w)
        cpp = pltpu.async_copy(pemb_h.at[idp_v], p_v, sem_p)
        cpw.wait()
        cpp.wait()

        def block_body(blk, _):
            tb = blk * TB
            ttv = idt_v[pl.ds(tb, TB)].astype(jnp.float32)
            ttf = [ttv[t] for t in range(TB)]

            def pass1(j, carry):
                s1 = list(carry[:TB])
                s2 = list(carry[TB:])
                sl = pl.ds(j * 16, 16)
                t0 = ty_v[0, sl]
                d = ty_v[1, sl] - t0
                for t in range(TB):
                    e = w_v[tb + t, sl] + p_v[tb + t, sl] + (t0 + ttf[t] * d)
                    w_v[tb + t, sl] = e
                    s1[t] = s1[t] + e
                    s2[t] = s2[t] + e * e
                return tuple(s1) + tuple(s2)

            zeros = jnp.zeros((16,), jnp.float32)
            carry = lax.fori_loop(0, NSL, pass1, (zeros,) * (2 * TB))

            for t in range(TB):
                s1_v[t, pl.ds(0, 16)] = carry[t]
                s2_v[t, pl.ds(0, 16)] = carry[TB + t]
            rows = jnp.arange(TB, dtype=jnp.int32)
            m = zeros
            q = zeros
            for l in range(16):
                li = jnp.full((16,), l, jnp.int32)
                m = m + plsc.load_gather(s1_v, [rows, li])
                q = q + plsc.load_gather(s2_v, [rows, li])
            muv = m * inv_h
            varv = q * inv_h - muv * muv + EPS
            rv = _rsqrt16(varv)
            mu = [muv[t] for t in range(TB)]
            rs = [rv[t] for t in range(TB)]

            def pass2(j, _):
                sl = pl.ds(j * 16, 16)
                g = g_v[sl]
                b = b_v[sl]
                for t in range(TB):
                    a = g * rs[t]
                    e = w_v[tb + t, sl]
                    w_v[tb + t, sl] = (e - mu[t]) * a + b
                return 0

            lax.fori_loop(0, NSL, pass2, 0)
            return 0

        lax.fori_loop(0, T_CHUNK // TB, block_body, 0)
        pltpu.sync_copy(w_v, out_h.at[pl.ds(off, T_CHUNK)])
        return 0

    lax.fori_loop(0, n_chunks, chunk_body, 0)


@jax.jit
def _run(ids, pids, tids, word_emb, pos_emb, type_emb, ln_gamma, ln_beta):
    total = ids.shape[0]
    mesh = plsc.VectorSubcoreMesh(core_axis_name="c", subcore_axis_name="s")
    k = pl.kernel(
        _sc_body,
        out_type=jax.ShapeDtypeStruct((total, HIDDEN), jnp.float32),
        mesh=mesh,
        compiler_params=pltpu.CompilerParams(needs_layout_passes=False),
        scratch_types=[
            pltpu.VMEM((T_CHUNK,), jnp.int32),
            pltpu.VMEM((T_CHUNK,), jnp.int32),
            pltpu.VMEM((T_CHUNK,), jnp.int32),
            pltpu.VMEM((T_CHUNK, HIDDEN), jnp.float32),
            pltpu.VMEM((T_CHUNK, HIDDEN), jnp.float32),
            pltpu.VMEM((2, HIDDEN), jnp.float32),
            pltpu.VMEM((HIDDEN,), jnp.float32),
            pltpu.VMEM((HIDDEN,), jnp.float32),
            pltpu.VMEM((TB, 16), jnp.float32),
            pltpu.VMEM((TB, 16), jnp.float32),
            pltpu.SemaphoreType.DMA,
            pltpu.SemaphoreType.DMA,
        ],
    )
    return k(ids, pids, tids, word_emb, pos_emb, type_emb, ln_gamma, ln_beta)


def kernel(input_ids, token_type_ids, position_ids, word_emb, pos_emb,
           type_emb, ln_gamma, ln_beta):
    bsz, seq = input_ids.shape
    out = _run(
        input_ids.reshape(-1),
        position_ids.reshape(-1),
        token_type_ids.reshape(-1),
        word_emb, pos_emb, type_emb, ln_gamma, ln_beta,
    )
    return out.reshape(bsz, seq, HIDDEN)


# speedup vs baseline: 2.0692x; 1.5120x over previous
import jax
import jax.numpy as jnp
from jax import lax
from jax.experimental import pallas as pl
from jax.experimental.pallas import tpu as pltpu
from jax.experimental.pallas import tpu_sc as plsc

HIDDEN = 768
NSL = HIDDEN // 16
EPS = 1e-12
T = 16
K_BUF = 4


def _rsqrt16(x):
    i = plsc.bitcast(x, jnp.int32)
    i = jnp.int32(0x5F3759DF) - (i >> 1)
    y = plsc.bitcast(i, jnp.float32)
    for _ in range(4):
        y = y * (1.5 - 0.5 * x * y * y)
    return y


def _sc_body(idall_h, wemb_h, pemb_h, temb_h, gam_h, bet_h, out_h,
             ids_v, w0, w1, w2, w3, p0, p1, p2, p3, ty_v, g_v, b_v,
             s1_v, s2_v, sg0, sg1, sg2, sg3, so0, so1, so2, so3):
    ws = (w0, w1, w2, w3)
    ps = (p0, p1, p2, p3)
    sg = (sg0, sg1, sg2, sg3)
    so = (so0, so1, so2, so3)

    info = plsc.get_sparse_core_info()
    nw = info.num_cores * info.num_subcores
    wid = lax.axis_index("s") * info.num_cores + lax.axis_index("c")
    total = idall_h.shape[1]
    per_w = total // nw
    n_ch = per_w // T
    base = wid * per_w
    inv_h = 1.0 / HIDDEN

    pltpu.sync_copy(idall_h.at[:, pl.ds(base, per_w)], ids_v)
    pltpu.sync_copy(temb_h, ty_v)
    pltpu.sync_copy(gam_h, g_v)
    pltpu.sync_copy(bet_h, b_v)

    def issue_gathers(c, b):
        offl = c * T
        pltpu.async_copy(wemb_h.at[ids_v.at[0, pl.ds(offl, T)]], ws[b], sg[b])
        pltpu.async_copy(pemb_h.at[ids_v.at[1, pl.ds(offl, T)]], ps[b], sg[b])

    for b in range(2):
        issue_gathers(b, b)

    def slot(c, b):
        offl = c * T
        off = base + offl
        pltpu.make_async_copy(
            wemb_h.at[ids_v.at[0, pl.ds(offl, T)]], ws[b], sg[b]).wait()
        pltpu.make_async_copy(
            pemb_h.at[ids_v.at[1, pl.ds(offl, T)]], ps[b], sg[b]).wait()

        ttv = ids_v[2, pl.ds(offl, T)].astype(jnp.float32)
        ttf = [ttv[t] for t in range(T)]
        w_v = ws[b]
        p_v = ps[b]

        def pass1(j, carry):
            s1 = list(carry[:T])
            s2 = list(carry[T:])
            sl = pl.ds(j * 16, 16)
            t0 = ty_v[0, sl]
            d = ty_v[1, sl] - t0
            for t in range(T):
                e = w_v[t, sl] + p_v[t, sl] + (t0 + ttf[t] * d)
                w_v[t, sl] = e
                s1[t] = s1[t] + e
                s2[t] = s2[t] + e * e
            return tuple(s1) + tuple(s2)

        zeros = jnp.zeros((16,), jnp.float32)
        carry = lax.fori_loop(0, NSL, pass1, (zeros,) * (2 * T))

        for t in range(T):
            s1_v[t, pl.ds(0, 16)] = carry[t]
            s2_v[t, pl.ds(0, 16)] = carry[T + t]
        rows = jnp.arange(T, dtype=jnp.int32)
        m = zeros
        q = zeros
        for l in range(16):
            li = jnp.full((16,), l, jnp.int32)
            m = m + plsc.load_gather(s1_v, [rows, li])
            q = q + plsc.load_gather(s2_v, [rows, li])
        muv = m * inv_h
        varv = q * inv_h - muv * muv + EPS
        rv = _rsqrt16(varv)
        mu = [muv[t] for t in range(T)]
        rs = [rv[t] for t in range(T)]

        def pass2(j, _):
            sl = pl.ds(j * 16, 16)
            g = g_v[sl]
            bb = b_v[sl]
            for t in range(T):
                a = g * rs[t]
                e = w_v[t, sl]
                w_v[t, sl] = (e - mu[t]) * a + bb
            return 0

        lax.fori_loop(0, NSL, pass2, 0)

        pltpu.async_copy(w_v, out_h.at[pl.ds(off, T)], so[b])

        n = c + 2
        bn = (b + 2) % K_BUF

        @pl.when(jnp.logical_and(n >= K_BUF, n < n_ch))
        def _():
            pltpu.make_async_copy(
                ws[bn], out_h.at[pl.ds(base + (n - K_BUF) * T, T)],
                so[bn]).wait()

        @pl.when(n < n_ch)
        def _():
            issue_gathers(n, bn)

        return 0

    def group(gi, _):
        for b in range(K_BUF):
            slot(gi * K_BUF + b, b)
        return 0

    lax.fori_loop(0, n_ch // K_BUF, group, 0)

    for b in range(K_BUF):
        pltpu.make_async_copy(
            ws[b], out_h.at[pl.ds(base + (n_ch - K_BUF + b) * T, T)],
            so[b]).wait()


@jax.jit
def _run(idall, word_emb, pos_emb, type_emb, ln_gamma, ln_beta):
    total = idall.shape[1]
    mesh = plsc.VectorSubcoreMesh(core_axis_name="c", subcore_axis_name="s")
    info = plsc.get_sparse_core_info()
    per_w = total // (info.num_cores * info.num_subcores)
    row_buf = lambda: pltpu.VMEM((T, HIDDEN), jnp.float32)
    k = pl.kernel(
        _sc_body,
        out_type=jax.ShapeDtypeStruct((total, HIDDEN), jnp.float32),
        mesh=mesh,
        compiler_params=pltpu.CompilerParams(needs_layout_passes=False),
        scratch_types=[
            pltpu.VMEM((3, per_w), jnp.int32),
            row_buf(), row_buf(), row_buf(), row_buf(),
            row_buf(), row_buf(), row_buf(), row_buf(),
            pltpu.VMEM((2, HIDDEN), jnp.float32),
            pltpu.VMEM((HIDDEN,), jnp.float32),
            pltpu.VMEM((HIDDEN,), jnp.float32),
            pltpu.VMEM((T, 16), jnp.float32),
            pltpu.VMEM((T, 16), jnp.float32),
        ] + [pltpu.SemaphoreType.DMA] * 8,
    )
    return k(idall, word_emb, pos_emb, type_emb, ln_gamma, ln_beta)


def kernel(input_ids, token_type_ids, position_ids, word_emb, pos_emb,
           type_emb, ln_gamma, ln_beta):
    bsz, seq = input_ids.shape
    idall = jnp.stack([
        input_ids.reshape(-1),
        position_ids.reshape(-1),
        token_type_ids.reshape(-1),
    ])
    out = _run(idall, word_emb, pos_emb, type_emb, ln_gamma, ln_beta)
    return out.reshape(bsz, seq, HIDDEN)
